# Initial kernel scaffold; baseline (speedup 1.0000x reference)
import jax
import jax.numpy as jnp
from jax.experimental import pallas as pl


def kernel(pred, connectivity, elem_directions, elem_lengths, prop_E, prop_A, line_load, bc_disp):
    raise NotImplementedError("write your pallas kernel here")


# speedup vs baseline: 48.7629x; 48.7629x over previous
import functools

import jax
import jax.numpy as jnp
from jax import lax
from jax.experimental import pallas as pl
from jax.experimental.pallas import tpu as pltpu
from jax.experimental.pallas import tpu_sc as plsc

N_NODES = 100000
N_ELEMS = 6400000
NPAD = 100352
CHUNK = 2048
NCHUNKS = N_ELEMS // CHUNK
NW = 32
SSPAN = NPAD // 16

_f32 = jnp.float32
_i32 = jnp.int32


def _sc_body(ux, uy, uz, ci, cj, dx, dy, dz, pE, pA, ln, fpart,
             bi, bj, bdx, bdy, bdz, bpE, bpA, bln,
             guix, guiy, guiz, gujx, gujy, gujz,
             fxp, fyp, fzp, fxn, fyn, fzn, zbuf,
             sux, suy, suz, sfx, sfy, sfz):
    c = lax.axis_index("c")
    s = lax.axis_index("s")
    w = s * 2 + c
    off = s * SSPAN

    def _zero(i, _):
        zbuf[pl.ds(i * 16, 16)] = jnp.zeros((16,), _f32)
        return 0
    lax.fori_loop(0, SSPAN // 16, _zero, 0)
    pltpu.sync_copy(zbuf, sfx.at[pl.ds(off, SSPAN)])
    pltpu.sync_copy(zbuf, sfy.at[pl.ds(off, SSPAN)])
    pltpu.sync_copy(zbuf, sfz.at[pl.ds(off, SSPAN)])
    for uc, su in ((ux, sux), (uy, suy), (uz, suz)):
        pltpu.sync_copy(uc.at[pl.ds(off, SSPAN)], zbuf)
        pltpu.sync_copy(zbuf, su.at[pl.ds(off, SSPAN)])
    plsc.subcore_barrier()

    n_iters = (NCHUNKS - w + NW - 1) // NW

    def _chunk(it, _):
        eb = (it * NW + w) * CHUNK
        pltpu.sync_copy(ci.at[pl.ds(eb, CHUNK)], bi)
        pltpu.sync_copy(cj.at[pl.ds(eb, CHUNK)], bj)
        pltpu.sync_copy(dx.at[pl.ds(eb, CHUNK)], bdx)
        pltpu.sync_copy(dy.at[pl.ds(eb, CHUNK)], bdy)
        pltpu.sync_copy(dz.at[pl.ds(eb, CHUNK)], bdz)
        pltpu.sync_copy(pE.at[pl.ds(eb, CHUNK)], bpE)
        pltpu.sync_copy(pA.at[pl.ds(eb, CHUNK)], bpA)
        pltpu.sync_copy(ln.at[pl.ds(eb, CHUNK)], bln)
        pltpu.sync_copy(sux.at[bi], guix)
        pltpu.sync_copy(suy.at[bi], guiy)
        pltpu.sync_copy(suz.at[bi], guiz)
        pltpu.sync_copy(sux.at[bj], gujx)
        pltpu.sync_copy(suy.at[bj], gujy)
        pltpu.sync_copy(suz.at[bj], gujz)

        def _step(k, _):
            sl = pl.ds(k * 16, 16)
            dux = gujx[sl] - guix[sl]
            duy = gujy[sl] - guiy[sl]
            duz = gujz[sl] - guiz[sl]
            dxv = bdx[sl]
            dyv = bdy[sl]
            dzv = bdz[sl]
            ax = dux * dxv + duy * dyv + duz * dzv
            f = bpE[sl] * bpA[sl] * ax / bln[sl]
            vx = f * dxv
            vy = f * dyv
            vz = f * dzv
            fxp[sl] = vx
            fyp[sl] = vy
            fzp[sl] = vz
            fxn[sl] = -vx
            fyn[sl] = -vy
            fzn[sl] = -vz
            return 0
        lax.fori_loop(0, CHUNK // 16, _step, 0)

        pltpu.sync_copy(fxp, sfx.at[bi], add=True)
        pltpu.sync_copy(fyp, sfy.at[bi], add=True)
        pltpu.sync_copy(fzp, sfz.at[bi], add=True)
        pltpu.sync_copy(fxn, sfx.at[bj], add=True)
        pltpu.sync_copy(fyn, sfy.at[bj], add=True)
        pltpu.sync_copy(fzn, sfz.at[bj], add=True)
        return 0

    lax.fori_loop(0, n_iters, _chunk, 0)

    plsc.subcore_barrier()
    for comp, sf in enumerate((sfx, sfy, sfz)):
        pltpu.sync_copy(sf.at[pl.ds(off, SSPAN)], zbuf)
        pltpu.sync_copy(zbuf, fpart.at[pl.ds((c * 3 + comp) * NPAD + off, SSPAN)])


_sc_call = functools.partial(
    pl.kernel,
    out_type=jax.ShapeDtypeStruct((6 * NPAD,), _f32),
    mesh=plsc.VectorSubcoreMesh(core_axis_name="c", subcore_axis_name="s"),
    scratch_types=[
        pltpu.VMEM((CHUNK,), _i32),
        pltpu.VMEM((CHUNK,), _i32),
        pltpu.VMEM((CHUNK,), _f32),
        pltpu.VMEM((CHUNK,), _f32),
        pltpu.VMEM((CHUNK,), _f32),
        pltpu.VMEM((CHUNK,), _f32),
        pltpu.VMEM((CHUNK,), _f32),
        pltpu.VMEM((CHUNK,), _f32),
        pltpu.VMEM((CHUNK,), _f32),
        pltpu.VMEM((CHUNK,), _f32),
        pltpu.VMEM((CHUNK,), _f32),
        pltpu.VMEM((CHUNK,), _f32),
        pltpu.VMEM((CHUNK,), _f32),
        pltpu.VMEM((CHUNK,), _f32),
        pltpu.VMEM((CHUNK,), _f32),
        pltpu.VMEM((CHUNK,), _f32),
        pltpu.VMEM((CHUNK,), _f32),
        pltpu.VMEM((CHUNK,), _f32),
        pltpu.VMEM((CHUNK,), _f32),
        pltpu.VMEM((CHUNK,), _f32),
        pltpu.VMEM((SSPAN,), _f32),
        pltpu.VMEM_SHARED((NPAD,), _f32),
        pltpu.VMEM_SHARED((NPAD,), _f32),
        pltpu.VMEM_SHARED((NPAD,), _f32),
        pltpu.VMEM_SHARED((NPAD,), _f32),
        pltpu.VMEM_SHARED((NPAD,), _f32),
        pltpu.VMEM_SHARED((NPAD,), _f32),
    ],
)(_sc_body)


def _loss_body(fp_ref, ll_ref, bc_ref, out_ref):
    r = fp_ref[0] + fp_ref[1] + ll_ref[...]
    free = bc_ref[...] < 0.5
    free3 = jnp.broadcast_to(free, r.shape)
    masked = jnp.where(free3, r * r, jnp.zeros_like(r))
    nfree = jnp.sum(jnp.where(free, 1.0, 0.0).astype(_f32))
    out_ref[0, 0] = jnp.sum(masked) / (nfree * 3.0)


_loss_call = pl.pallas_call(
    _loss_body,
    out_shape=jax.ShapeDtypeStruct((1, 1), _f32),
    out_specs=pl.BlockSpec(memory_space=pltpu.SMEM),
)


def kernel(pred, connectivity, elem_directions, elem_lengths, prop_E, prop_A,
           line_load, bc_disp):
    padn = NPAD - N_NODES
    ux = jnp.pad(pred[:, 0], (0, padn))
    uy = jnp.pad(pred[:, 1], (0, padn))
    uz = jnp.pad(pred[:, 2], (0, padn))
    conn = connectivity.astype(_i32)
    ci = conn[:, 0]
    cj = conn[:, 1]
    dx = elem_directions[:, 0]
    dy = elem_directions[:, 1]
    dz = elem_directions[:, 2]
    fpart = _sc_call(ux, uy, uz, ci, cj, dx, dy, dz, prop_E, prop_A,
                     elem_lengths).reshape(2, 3, NPAD)
    llt = jnp.pad(line_load.T, ((0, 0), (0, padn)))
    bcp = jnp.pad(bc_disp[:, 0], (0, padn), constant_values=1.0).reshape(1, NPAD)
    loss2d = _loss_call(fpart, llt, bcp)
    return loss2d[0, 0]


# speedup vs baseline: 69.9846x; 1.4352x over previous
import functools

import jax
import jax.numpy as jnp
from jax import lax
from jax.experimental import pallas as pl
from jax.experimental.pallas import tpu as pltpu
from jax.experimental.pallas import tpu_sc as plsc

N_NODES = 100000
N_ELEMS = 6400000
NPAD = 100352
CHUNK = 2048
NCHUNKS = N_ELEMS // CHUNK
NW = 32
SSPAN = NPAD // 16

_f32 = jnp.float32
_i32 = jnp.int32


def _sc_body(ux, uy, uz, ci, cj, dx, dy, dz, pE, pA, ln, fpart,
             bi, bj, bdx, bdy, bdz, bpE, bpA, bln,
             guix, guiy, guiz, gujx, gujy, gujz,
             fxp, fyp, fzp, fxn, fyn, fzn, zbuf,
             sem_l, sem_g, sem_s,
             sux, suy, suz, sfx, sfy, sfz):
    c = lax.axis_index("c")
    s = lax.axis_index("s")
    w = s * 2 + c
    off = s * SSPAN

    def _zero(i, _):
        zbuf[pl.ds(i * 16, 16)] = jnp.zeros((16,), _f32)
        return 0
    lax.fori_loop(0, SSPAN // 16, _zero, 0)
    pltpu.sync_copy(zbuf, sfx.at[pl.ds(off, SSPAN)])
    pltpu.sync_copy(zbuf, sfy.at[pl.ds(off, SSPAN)])
    pltpu.sync_copy(zbuf, sfz.at[pl.ds(off, SSPAN)])
    for uc, su in ((ux, sux), (uy, suy), (uz, suz)):
        pltpu.sync_copy(uc.at[pl.ds(off, SSPAN)], zbuf)
        pltpu.sync_copy(zbuf, su.at[pl.ds(off, SSPAN)])
    plsc.subcore_barrier()

    n_iters = (NCHUNKS - w + NW - 1) // NW

    def _chunk(it, _):
        eb = (it * NW + w) * CHUNK
        d_idx = [
            pltpu.async_copy(ci.at[pl.ds(eb, CHUNK)], bi, sem_l),
            pltpu.async_copy(cj.at[pl.ds(eb, CHUNK)], bj, sem_l),
        ]
        d_lin = [
            pltpu.async_copy(dx.at[pl.ds(eb, CHUNK)], bdx, sem_l),
            pltpu.async_copy(dy.at[pl.ds(eb, CHUNK)], bdy, sem_l),
            pltpu.async_copy(dz.at[pl.ds(eb, CHUNK)], bdz, sem_l),
            pltpu.async_copy(pE.at[pl.ds(eb, CHUNK)], bpE, sem_l),
            pltpu.async_copy(pA.at[pl.ds(eb, CHUNK)], bpA, sem_l),
            pltpu.async_copy(ln.at[pl.ds(eb, CHUNK)], bln, sem_l),
        ]
        for d in d_idx:
            d.wait()
        d_g = [
            pltpu.async_copy(sux.at[bi], guix, sem_g),
            pltpu.async_copy(suy.at[bi], guiy, sem_g),
            pltpu.async_copy(suz.at[bi], guiz, sem_g),
            pltpu.async_copy(sux.at[bj], gujx, sem_g),
            pltpu.async_copy(suy.at[bj], gujy, sem_g),
            pltpu.async_copy(suz.at[bj], gujz, sem_g),
        ]
        for d in d_lin:
            d.wait()
        for d in d_g:
            d.wait()

        def _step(k, _):
            sl = pl.ds(k * 16, 16)
            dux = gujx[sl] - guix[sl]
            duy = gujy[sl] - guiy[sl]
            duz = gujz[sl] - guiz[sl]
            dxv = bdx[sl]
            dyv = bdy[sl]
            dzv = bdz[sl]
            ax = dux * dxv + duy * dyv + duz * dzv
            f = bpE[sl] * bpA[sl] * ax / bln[sl]
            vx = f * dxv
            vy = f * dyv
            vz = f * dzv
            fxp[sl] = vx
            fyp[sl] = vy
            fzp[sl] = vz
            fxn[sl] = -vx
            fyn[sl] = -vy
            fzn[sl] = -vz
            return 0
        lax.fori_loop(0, CHUNK // 16, _step, 0)

        d_s = [
            pltpu.async_copy(fxp, sfx.at[bi], sem_s, add=True),
            pltpu.async_copy(fyp, sfy.at[bi], sem_s, add=True),
            pltpu.async_copy(fzp, sfz.at[bi], sem_s, add=True),
            pltpu.async_copy(fxn, sfx.at[bj], sem_s, add=True),
            pltpu.async_copy(fyn, sfy.at[bj], sem_s, add=True),
            pltpu.async_copy(fzn, sfz.at[bj], sem_s, add=True),
        ]
        for d in d_s:
            d.wait()
        return 0

    lax.fori_loop(0, n_iters, _chunk, 0)

    plsc.subcore_barrier()
    for comp, sf in enumerate((sfx, sfy, sfz)):
        pltpu.sync_copy(sf.at[pl.ds(off, SSPAN)], zbuf)
        pltpu.sync_copy(zbuf, fpart.at[pl.ds((c * 3 + comp) * NPAD + off, SSPAN)])


_sc_call = functools.partial(
    pl.kernel,
    out_type=jax.ShapeDtypeStruct((6 * NPAD,), _f32),
    mesh=plsc.VectorSubcoreMesh(core_axis_name="c", subcore_axis_name="s"),
    scratch_types=[
        pltpu.VMEM((CHUNK,), _i32),
        pltpu.VMEM((CHUNK,), _i32),
        pltpu.VMEM((CHUNK,), _f32),
        pltpu.VMEM((CHUNK,), _f32),
        pltpu.VMEM((CHUNK,), _f32),
        pltpu.VMEM((CHUNK,), _f32),
        pltpu.VMEM((CHUNK,), _f32),
        pltpu.VMEM((CHUNK,), _f32),
        pltpu.VMEM((CHUNK,), _f32),
        pltpu.VMEM((CHUNK,), _f32),
        pltpu.VMEM((CHUNK,), _f32),
        pltpu.VMEM((CHUNK,), _f32),
        pltpu.VMEM((CHUNK,), _f32),
        pltpu.VMEM((CHUNK,), _f32),
        pltpu.VMEM((CHUNK,), _f32),
        pltpu.VMEM((CHUNK,), _f32),
        pltpu.VMEM((CHUNK,), _f32),
        pltpu.VMEM((CHUNK,), _f32),
        pltpu.VMEM((CHUNK,), _f32),
        pltpu.VMEM((CHUNK,), _f32),
        pltpu.VMEM((SSPAN,), _f32),
        pltpu.SemaphoreType.DMA,
        pltpu.SemaphoreType.DMA,
        pltpu.SemaphoreType.DMA,
        pltpu.VMEM_SHARED((NPAD,), _f32),
        pltpu.VMEM_SHARED((NPAD,), _f32),
        pltpu.VMEM_SHARED((NPAD,), _f32),
        pltpu.VMEM_SHARED((NPAD,), _f32),
        pltpu.VMEM_SHARED((NPAD,), _f32),
        pltpu.VMEM_SHARED((NPAD,), _f32),
    ],
)(_sc_body)


def _loss_body(fp_ref, ll_ref, bc_ref, out_ref):
    r = fp_ref[0] + fp_ref[1] + ll_ref[...]
    free = bc_ref[...] < 0.5
    free3 = jnp.broadcast_to(free, r.shape)
    masked = jnp.where(free3, r * r, jnp.zeros_like(r))
    nfree = jnp.sum(jnp.where(free, 1.0, 0.0).astype(_f32))
    out_ref[0, 0] = jnp.sum(masked) / (nfree * 3.0)


_loss_call = pl.pallas_call(
    _loss_body,
    out_shape=jax.ShapeDtypeStruct((1, 1), _f32),
    out_specs=pl.BlockSpec(memory_space=pltpu.SMEM),
)


def kernel(pred, connectivity, elem_directions, elem_lengths, prop_E, prop_A,
           line_load, bc_disp):
    padn = NPAD - N_NODES
    ux = jnp.pad(pred[:, 0], (0, padn))
    uy = jnp.pad(pred[:, 1], (0, padn))
    uz = jnp.pad(pred[:, 2], (0, padn))
    conn = connectivity.astype(_i32)
    ci = conn[:, 0]
    cj = conn[:, 1]
    dx = elem_directions[:, 0]
    dy = elem_directions[:, 1]
    dz = elem_directions[:, 2]
    fpart = _sc_call(ux, uy, uz, ci, cj, dx, dy, dz, prop_E, prop_A,
                     elem_lengths).reshape(2, 3, NPAD)
    llt = jnp.pad(line_load.T, ((0, 0), (0, padn)))
    bcp = jnp.pad(bc_disp[:, 0], (0, padn), constant_values=1.0).reshape(1, NPAD)
    loss2d = _loss_call(fpart, llt, bcp)
    return loss2d[0, 0]


# speedup vs baseline: 74.5667x; 1.0655x over previous
import functools

import jax
import jax.numpy as jnp
from jax import lax
from jax.experimental import pallas as pl
from jax.experimental.pallas import tpu as pltpu
from jax.experimental.pallas import tpu_sc as plsc

N_NODES = 100000
N_ELEMS = 6400000
NPAD = 100352
CHUNK = 4000
NCHUNKS = N_ELEMS // CHUNK
NW = 32
SSPAN = NPAD // 16

_f32 = jnp.float32
_i32 = jnp.int32


def _sc_body(ux, uy, uz, ci, cj, dx, dy, dz, pE, pA, ln, fpart,
             bi, bj, bdx, bdy, bdz, bpE, bpA, bln,
             guix, guiy, guiz, gujx, gujy, gujz,
             fxp, fyp, fzp, fxn, fyn, fzn, zbuf,
             sem_l, sem_g, sem_s,
             sux, suy, suz, sfx, sfy, sfz):
    c = lax.axis_index("c")
    s = lax.axis_index("s")
    w = s * 2 + c
    off = s * SSPAN

    def _zero(i, _):
        zbuf[pl.ds(i * 16, 16)] = jnp.zeros((16,), _f32)
        return 0
    lax.fori_loop(0, SSPAN // 16, _zero, 0)
    pltpu.sync_copy(zbuf, sfx.at[pl.ds(off, SSPAN)])
    pltpu.sync_copy(zbuf, sfy.at[pl.ds(off, SSPAN)])
    pltpu.sync_copy(zbuf, sfz.at[pl.ds(off, SSPAN)])
    for uc, su in ((ux, sux), (uy, suy), (uz, suz)):
        pltpu.sync_copy(uc.at[pl.ds(off, SSPAN)], zbuf)
        pltpu.sync_copy(zbuf, su.at[pl.ds(off, SSPAN)])
    plsc.subcore_barrier()

    n_iters = (NCHUNKS - w + NW - 1) // NW

    def _chunk(it, _):
        eb = (it * NW + w) * CHUNK
        d_idx = [
            pltpu.async_copy(ci.at[pl.ds(eb, CHUNK)], bi, sem_l),
            pltpu.async_copy(cj.at[pl.ds(eb, CHUNK)], bj, sem_l),
        ]
        d_lin = [
            pltpu.async_copy(dx.at[pl.ds(eb, CHUNK)], bdx, sem_l),
            pltpu.async_copy(dy.at[pl.ds(eb, CHUNK)], bdy, sem_l),
            pltpu.async_copy(dz.at[pl.ds(eb, CHUNK)], bdz, sem_l),
            pltpu.async_copy(pE.at[pl.ds(eb, CHUNK)], bpE, sem_l),
            pltpu.async_copy(pA.at[pl.ds(eb, CHUNK)], bpA, sem_l),
            pltpu.async_copy(ln.at[pl.ds(eb, CHUNK)], bln, sem_l),
        ]
        for d in d_idx:
            d.wait()
        d_g = [
            pltpu.async_copy(sux.at[bi], guix, sem_g),
            pltpu.async_copy(suy.at[bi], guiy, sem_g),
            pltpu.async_copy(suz.at[bi], guiz, sem_g),
            pltpu.async_copy(sux.at[bj], gujx, sem_g),
            pltpu.async_copy(suy.at[bj], gujy, sem_g),
            pltpu.async_copy(suz.at[bj], gujz, sem_g),
        ]
        for d in d_lin:
            d.wait()
        for d in d_g:
            d.wait()

        def _step(k, _):
            sl = pl.ds(k * 16, 16)
            dux = gujx[sl] - guix[sl]
            duy = gujy[sl] - guiy[sl]
            duz = gujz[sl] - guiz[sl]
            dxv = bdx[sl]
            dyv = bdy[sl]
            dzv = bdz[sl]
            ax = dux * dxv + duy * dyv + duz * dzv
            f = bpE[sl] * bpA[sl] * ax / bln[sl]
            vx = f * dxv
            vy = f * dyv
            vz = f * dzv
            fxp[sl] = vx
            fyp[sl] = vy
            fzp[sl] = vz
            fxn[sl] = -vx
            fyn[sl] = -vy
            fzn[sl] = -vz
            return 0
        lax.fori_loop(0, CHUNK // 16, _step, 0)

        d_s = [
            pltpu.async_copy(fxp, sfx.at[bi], sem_s, add=True),
            pltpu.async_copy(fyp, sfy.at[bi], sem_s, add=True),
            pltpu.async_copy(fzp, sfz.at[bi], sem_s, add=True),
            pltpu.async_copy(fxn, sfx.at[bj], sem_s, add=True),
            pltpu.async_copy(fyn, sfy.at[bj], sem_s, add=True),
            pltpu.async_copy(fzn, sfz.at[bj], sem_s, add=True),
        ]
        for d in d_s:
            d.wait()
        return 0

    lax.fori_loop(0, n_iters, _chunk, 0)

    plsc.subcore_barrier()
    for comp, sf in enumerate((sfx, sfy, sfz)):
        pltpu.sync_copy(sf.at[pl.ds(off, SSPAN)], zbuf)
        pltpu.sync_copy(zbuf, fpart.at[pl.ds((c * 3 + comp) * NPAD + off, SSPAN)])


_sc_call = functools.partial(
    pl.kernel,
    out_type=jax.ShapeDtypeStruct((6 * NPAD,), _f32),
    mesh=plsc.VectorSubcoreMesh(core_axis_name="c", subcore_axis_name="s"),
    scratch_types=[
        pltpu.VMEM((CHUNK,), _i32),
        pltpu.VMEM((CHUNK,), _i32),
        pltpu.VMEM((CHUNK,), _f32),
        pltpu.VMEM((CHUNK,), _f32),
        pltpu.VMEM((CHUNK,), _f32),
        pltpu.VMEM((CHUNK,), _f32),
        pltpu.VMEM((CHUNK,), _f32),
        pltpu.VMEM((CHUNK,), _f32),
        pltpu.VMEM((CHUNK,), _f32),
        pltpu.VMEM((CHUNK,), _f32),
        pltpu.VMEM((CHUNK,), _f32),
        pltpu.VMEM((CHUNK,), _f32),
        pltpu.VMEM((CHUNK,), _f32),
        pltpu.VMEM((CHUNK,), _f32),
        pltpu.VMEM((CHUNK,), _f32),
        pltpu.VMEM((CHUNK,), _f32),
        pltpu.VMEM((CHUNK,), _f32),
        pltpu.VMEM((CHUNK,), _f32),
        pltpu.VMEM((CHUNK,), _f32),
        pltpu.VMEM((CHUNK,), _f32),
        pltpu.VMEM((SSPAN,), _f32),
        pltpu.SemaphoreType.DMA,
        pltpu.SemaphoreType.DMA,
        pltpu.SemaphoreType.DMA,
        pltpu.VMEM_SHARED((NPAD,), _f32),
        pltpu.VMEM_SHARED((NPAD,), _f32),
        pltpu.VMEM_SHARED((NPAD,), _f32),
        pltpu.VMEM_SHARED((NPAD,), _f32),
        pltpu.VMEM_SHARED((NPAD,), _f32),
        pltpu.VMEM_SHARED((NPAD,), _f32),
    ],
)(_sc_body)


def _loss_body(fp_ref, ll_ref, bc_ref, out_ref):
    r = fp_ref[0] + fp_ref[1] + ll_ref[...]
    free = bc_ref[...] < 0.5
    free3 = jnp.broadcast_to(free, r.shape)
    masked = jnp.where(free3, r * r, jnp.zeros_like(r))
    nfree = jnp.sum(jnp.where(free, 1.0, 0.0).astype(_f32))
    out_ref[0, 0] = jnp.sum(masked) / (nfree * 3.0)


_loss_call = pl.pallas_call(
    _loss_body,
    out_shape=jax.ShapeDtypeStruct((1, 1), _f32),
    out_specs=pl.BlockSpec(memory_space=pltpu.SMEM),
)


def kernel(pred, connectivity, elem_directions, elem_lengths, prop_E, prop_A,
           line_load, bc_disp):
    padn = NPAD - N_NODES
    ux = jnp.pad(pred[:, 0], (0, padn))
    uy = jnp.pad(pred[:, 1], (0, padn))
    uz = jnp.pad(pred[:, 2], (0, padn))
    conn = connectivity.astype(_i32)
    ci = conn[:, 0]
    cj = conn[:, 1]
    dx = elem_directions[:, 0]
    dy = elem_directions[:, 1]
    dz = elem_directions[:, 2]
    fpart = _sc_call(ux, uy, uz, ci, cj, dx, dy, dz, prop_E, prop_A,
                     elem_lengths).reshape(2, 3, NPAD)
    llt = jnp.pad(line_load.T, ((0, 0), (0, padn)))
    bcp = jnp.pad(bc_disp[:, 0], (0, padn), constant_values=1.0).reshape(1, NPAD)
    loss2d = _loss_call(fpart, llt, bcp)
    return loss2d[0, 0]


# speedup vs baseline: 78.4111x; 1.0516x over previous
import functools

import jax
import jax.numpy as jnp
from jax import lax
from jax.experimental import pallas as pl
from jax.experimental.pallas import tpu as pltpu
from jax.experimental.pallas import tpu_sc as plsc

N_NODES = 100000
N_ELEMS = 6400000
NPAD = 100352
CHUNK = 2000
NCHUNKS = N_ELEMS // CHUNK
NW = 32
SSPAN = NPAD // 16

_f32 = jnp.float32
_i32 = jnp.int32


def _sc_body(ux, uy, uz, ci, cj, dx, dy, dz, pE, pA, ln, fpart,
             bi, bj, bi2, bj2, bdx, bdy, bdz, bpE, bpA, bln,
             guix, guiy, guiz, gujx, gujy, gujz,
             fxp, fyp, fzp, fxn, fyn, fzn,
             fxp2, fyp2, fzp2, fxn2, fyn2, fzn2, zbuf,
             sem_l, sem_g, sem_s,
             sux, suy, suz, sfx, sfy, sfz):
    c = lax.axis_index("c")
    s = lax.axis_index("s")
    w = s * 2 + c
    off = s * SSPAN

    def _zero(i, _):
        zbuf[pl.ds(i * 16, 16)] = jnp.zeros((16,), _f32)
        return 0
    lax.fori_loop(0, SSPAN // 16, _zero, 0)
    pltpu.sync_copy(zbuf, sfx.at[pl.ds(off, SSPAN)])
    pltpu.sync_copy(zbuf, sfy.at[pl.ds(off, SSPAN)])
    pltpu.sync_copy(zbuf, sfz.at[pl.ds(off, SSPAN)])
    for uc, su in ((ux, sux), (uy, suy), (uz, suz)):
        pltpu.sync_copy(uc.at[pl.ds(off, SSPAN)], zbuf)
        pltpu.sync_copy(zbuf, su.at[pl.ds(off, SSPAN)])
    plsc.subcore_barrier()

    n_pairs = NCHUNKS // NW // 2

    def _phase(eb, mbi, mbj, mfxp, mfyp, mfzp, mfxn, mfyn, mfzn,
               prev_scatter):
        d_idx = [
            pltpu.async_copy(ci.at[pl.ds(eb, CHUNK)], mbi, sem_l),
            pltpu.async_copy(cj.at[pl.ds(eb, CHUNK)], mbj, sem_l),
        ]
        d_lin = [
            pltpu.async_copy(dx.at[pl.ds(eb, CHUNK)], bdx, sem_l),
            pltpu.async_copy(dy.at[pl.ds(eb, CHUNK)], bdy, sem_l),
            pltpu.async_copy(dz.at[pl.ds(eb, CHUNK)], bdz, sem_l),
            pltpu.async_copy(pE.at[pl.ds(eb, CHUNK)], bpE, sem_l),
            pltpu.async_copy(pA.at[pl.ds(eb, CHUNK)], bpA, sem_l),
            pltpu.async_copy(ln.at[pl.ds(eb, CHUNK)], bln, sem_l),
        ]
        for d in d_idx:
            d.wait()
        d_g = [
            pltpu.async_copy(sux.at[mbi], guix, sem_g),
            pltpu.async_copy(suy.at[mbi], guiy, sem_g),
            pltpu.async_copy(suz.at[mbi], guiz, sem_g),
            pltpu.async_copy(sux.at[mbj], gujx, sem_g),
            pltpu.async_copy(suy.at[mbj], gujy, sem_g),
            pltpu.async_copy(suz.at[mbj], gujz, sem_g),
        ]
        for d in d_lin:
            d.wait()
        for d in d_g:
            d.wait()

        def _step(k, _):
            sl = pl.ds(k * 16, 16)
            dux = gujx[sl] - guix[sl]
            duy = gujy[sl] - guiy[sl]
            duz = gujz[sl] - guiz[sl]
            dxv = bdx[sl]
            dyv = bdy[sl]
            dzv = bdz[sl]
            ax = dux * dxv + duy * dyv + duz * dzv
            f = bpE[sl] * bpA[sl] * ax / bln[sl]
            vx = f * dxv
            vy = f * dyv
            vz = f * dzv
            mfxp[sl] = vx
            mfyp[sl] = vy
            mfzp[sl] = vz
            mfxn[sl] = -vx
            mfyn[sl] = -vy
            mfzn[sl] = -vz
            return 0
        lax.fori_loop(0, CHUNK // 16, _step, 0)

        for d in prev_scatter:
            d.wait()
        return [
            pltpu.async_copy(mfxp, sfx.at[mbi], sem_s, add=True),
            pltpu.async_copy(mfyp, sfy.at[mbi], sem_s, add=True),
            pltpu.async_copy(mfzp, sfz.at[mbi], sem_s, add=True),
            pltpu.async_copy(mfxn, sfx.at[mbj], sem_s, add=True),
            pltpu.async_copy(mfyn, sfy.at[mbj], sem_s, add=True),
            pltpu.async_copy(mfzn, sfz.at[mbj], sem_s, add=True),
        ]

    def _pair(p, _):
        eb_a = ((2 * p) * NW + w) * CHUNK
        eb_b = ((2 * p + 1) * NW + w) * CHUNK
        ds_a = _phase(eb_a, bi, bj, fxp, fyp, fzp, fxn, fyn, fzn, [])
        ds_b = _phase(eb_b, bi2, bj2, fxp2, fyp2, fzp2, fxn2, fyn2, fzn2,
                      ds_a)
        for d in ds_b:
            d.wait()
        return 0

    lax.fori_loop(0, n_pairs, _pair, 0)

    plsc.subcore_barrier()
    for comp, sf in enumerate((sfx, sfy, sfz)):
        pltpu.sync_copy(sf.at[pl.ds(off, SSPAN)], zbuf)
        pltpu.sync_copy(zbuf, fpart.at[pl.ds((c * 3 + comp) * NPAD + off, SSPAN)])


_sc_call = functools.partial(
    pl.kernel,
    out_type=jax.ShapeDtypeStruct((6 * NPAD,), _f32),
    mesh=plsc.VectorSubcoreMesh(core_axis_name="c", subcore_axis_name="s"),
    scratch_types=[
        pltpu.VMEM((CHUNK,), _i32),
        pltpu.VMEM((CHUNK,), _i32),
        pltpu.VMEM((CHUNK,), _i32),
        pltpu.VMEM((CHUNK,), _i32),
        pltpu.VMEM((CHUNK,), _f32),
        pltpu.VMEM((CHUNK,), _f32),
        pltpu.VMEM((CHUNK,), _f32),
        pltpu.VMEM((CHUNK,), _f32),
        pltpu.VMEM((CHUNK,), _f32),
        pltpu.VMEM((CHUNK,), _f32),
        pltpu.VMEM((CHUNK,), _f32),
        pltpu.VMEM((CHUNK,), _f32),
        pltpu.VMEM((CHUNK,), _f32),
        pltpu.VMEM((CHUNK,), _f32),
        pltpu.VMEM((CHUNK,), _f32),
        pltpu.VMEM((CHUNK,), _f32),
        pltpu.VMEM((CHUNK,), _f32),
        pltpu.VMEM((CHUNK,), _f32),
        pltpu.VMEM((CHUNK,), _f32),
        pltpu.VMEM((CHUNK,), _f32),
        pltpu.VMEM((CHUNK,), _f32),
        pltpu.VMEM((CHUNK,), _f32),
        pltpu.VMEM((CHUNK,), _f32),
        pltpu.VMEM((CHUNK,), _f32),
        pltpu.VMEM((CHUNK,), _f32),
        pltpu.VMEM((CHUNK,), _f32),
        pltpu.VMEM((CHUNK,), _f32),
        pltpu.VMEM((CHUNK,), _f32),
        pltpu.VMEM((SSPAN,), _f32),
        pltpu.SemaphoreType.DMA,
        pltpu.SemaphoreType.DMA,
        pltpu.SemaphoreType.DMA,
        pltpu.VMEM_SHARED((NPAD,), _f32),
        pltpu.VMEM_SHARED((NPAD,), _f32),
        pltpu.VMEM_SHARED((NPAD,), _f32),
        pltpu.VMEM_SHARED((NPAD,), _f32),
        pltpu.VMEM_SHARED((NPAD,), _f32),
        pltpu.VMEM_SHARED((NPAD,), _f32),
    ],
)(_sc_body)


def _loss_body(fp_ref, ll_ref, bc_ref, out_ref):
    r = fp_ref[0] + fp_ref[1] + ll_ref[...]
    free = bc_ref[...] < 0.5
    free3 = jnp.broadcast_to(free, r.shape)
    masked = jnp.where(free3, r * r, jnp.zeros_like(r))
    nfree = jnp.sum(jnp.where(free, 1.0, 0.0).astype(_f32))
    out_ref[0, 0] = jnp.sum(masked) / (nfree * 3.0)


_loss_call = pl.pallas_call(
    _loss_body,
    out_shape=jax.ShapeDtypeStruct((1, 1), _f32),
    out_specs=pl.BlockSpec(memory_space=pltpu.SMEM),
)


def kernel(pred, connectivity, elem_directions, elem_lengths, prop_E, prop_A,
           line_load, bc_disp):
    padn = NPAD - N_NODES
    ux = jnp.pad(pred[:, 0], (0, padn))
    uy = jnp.pad(pred[:, 1], (0, padn))
    uz = jnp.pad(pred[:, 2], (0, padn))
    conn = connectivity.astype(_i32)
    ci = conn[:, 0]
    cj = conn[:, 1]
    dx = elem_directions[:, 0]
    dy = elem_directions[:, 1]
    dz = elem_directions[:, 2]
    fpart = _sc_call(ux, uy, uz, ci, cj, dx, dy, dz, prop_E, prop_A,
                     elem_lengths).reshape(2, 3, NPAD)
    llt = jnp.pad(line_load.T, ((0, 0), (0, padn)))
    bcp = jnp.pad(bc_disp[:, 0], (0, padn), constant_values=1.0).reshape(1, NPAD)
    loss2d = _loss_call(fpart, llt, bcp)
    return loss2d[0, 0]


# speedup vs baseline: 80.1477x; 1.0221x over previous
import functools

import jax
import jax.numpy as jnp
from jax import lax
from jax.experimental import pallas as pl
from jax.experimental.pallas import tpu as pltpu
from jax.experimental.pallas import tpu_sc as plsc

N_NODES = 100000
N_ELEMS = 6400000
NPAD = 100352
CHUNK = 2000
NCHUNKS = N_ELEMS // CHUNK
NW = 32
SSPAN = NPAD // 16

_f32 = jnp.float32
_i32 = jnp.int32


def _sc_body(ux, uy, uz, ci, cj, dx, dy, dz, pE, pA, ln, fpart,
             bi, bj, bi2, bj2, bdx, bdy, bdz, bpE, bpA, bln,
             guix, guiy, guiz, gujx, gujy, gujz,
             fxp, fyp, fzp, fxn, fyn, fzn,
             fxp2, fyp2, fzp2, fxn2, fyn2, fzn2, zbuf,
             sem_l, sem_g, sem_s,
             sux, suy, suz, sfx, sfy, sfz):
    c = lax.axis_index("c")
    s = lax.axis_index("s")
    w = s * 2 + c
    off = s * SSPAN

    def _zero(i, _):
        zbuf[pl.ds(i * 16, 16)] = jnp.zeros((16,), _f32)
        return 0
    lax.fori_loop(0, SSPAN // 16, _zero, 0)
    pltpu.sync_copy(zbuf, sfx.at[pl.ds(off, SSPAN)])
    pltpu.sync_copy(zbuf, sfy.at[pl.ds(off, SSPAN)])
    pltpu.sync_copy(zbuf, sfz.at[pl.ds(off, SSPAN)])
    for uc, su in ((ux, sux), (uy, suy), (uz, suz)):
        pltpu.sync_copy(uc.at[pl.ds(off, SSPAN)], zbuf)
        pltpu.sync_copy(zbuf, su.at[pl.ds(off, SSPAN)])
    plsc.subcore_barrier()

    def _zero2(i, _):
        zbuf[pl.ds(i * 16, 16)] = jnp.zeros((16,), _f32)
        return 0
    lax.fori_loop(0, CHUNK // 16, _zero2, 0)
    pltpu.sync_copy(ci.at[pl.ds(w * CHUNK, CHUNK)], bi2)
    zsrc = zbuf.at[pl.ds(0, CHUNK)]
    pltpu.async_copy(zsrc, sfx.at[bi2], sem_s, add=True)
    pltpu.async_copy(zsrc, sfy.at[bi2], sem_s, add=True)
    pltpu.async_copy(zsrc, sfz.at[bi2], sem_s, add=True)
    pltpu.async_copy(zsrc, sfx.at[bi2], sem_s, add=True)
    pltpu.async_copy(zsrc, sfy.at[bi2], sem_s, add=True)
    pltpu.async_copy(zsrc, sfz.at[bi2], sem_s, add=True)

    n_pairs = NCHUNKS // NW // 2

    def _phase(eb, mbi, mbj, mfxp, mfyp, mfzp, mfxn, mfyn, mfzn,
               prev_scatter):
        d_idx = [
            pltpu.async_copy(ci.at[pl.ds(eb, CHUNK)], mbi, sem_l),
            pltpu.async_copy(cj.at[pl.ds(eb, CHUNK)], mbj, sem_l),
        ]
        d_lin = [
            pltpu.async_copy(dx.at[pl.ds(eb, CHUNK)], bdx, sem_l),
            pltpu.async_copy(dy.at[pl.ds(eb, CHUNK)], bdy, sem_l),
            pltpu.async_copy(dz.at[pl.ds(eb, CHUNK)], bdz, sem_l),
            pltpu.async_copy(pE.at[pl.ds(eb, CHUNK)], bpE, sem_l),
            pltpu.async_copy(pA.at[pl.ds(eb, CHUNK)], bpA, sem_l),
            pltpu.async_copy(ln.at[pl.ds(eb, CHUNK)], bln, sem_l),
        ]
        for d in d_idx:
            d.wait()
        d_g = [
            pltpu.async_copy(sux.at[mbi], guix, sem_g),
            pltpu.async_copy(suy.at[mbi], guiy, sem_g),
            pltpu.async_copy(suz.at[mbi], guiz, sem_g),
            pltpu.async_copy(sux.at[mbj], gujx, sem_g),
            pltpu.async_copy(suy.at[mbj], gujy, sem_g),
            pltpu.async_copy(suz.at[mbj], gujz, sem_g),
        ]
        for d in d_lin:
            d.wait()
        for d in d_g:
            d.wait()

        def _step(k, _):
            sl = pl.ds(k * 16, 16)
            dux = gujx[sl] - guix[sl]
            duy = gujy[sl] - guiy[sl]
            duz = gujz[sl] - guiz[sl]
            dxv = bdx[sl]
            dyv = bdy[sl]
            dzv = bdz[sl]
            ax = dux * dxv + duy * dyv + duz * dzv
            f = bpE[sl] * bpA[sl] * ax / bln[sl]
            vx = f * dxv
            vy = f * dyv
            vz = f * dzv
            mfxp[sl] = vx
            mfyp[sl] = vy
            mfzp[sl] = vz
            mfxn[sl] = -vx
            mfyn[sl] = -vy
            mfzn[sl] = -vz
            return 0
        lax.fori_loop(0, CHUNK // 16, _step, 0)

        for d in prev_scatter:
            d.wait()
        return [
            pltpu.async_copy(mfxp, sfx.at[mbi], sem_s, add=True),
            pltpu.async_copy(mfyp, sfy.at[mbi], sem_s, add=True),
            pltpu.async_copy(mfzp, sfz.at[mbi], sem_s, add=True),
            pltpu.async_copy(mfxn, sfx.at[mbj], sem_s, add=True),
            pltpu.async_copy(mfyn, sfy.at[mbj], sem_s, add=True),
            pltpu.async_copy(mfzn, sfz.at[mbj], sem_s, add=True),
        ]

    def _prev_b_waits():
        return [
            pltpu.make_async_copy(fxp2, sfx.at[bi2], sem_s),
            pltpu.make_async_copy(fyp2, sfy.at[bi2], sem_s),
            pltpu.make_async_copy(fzp2, sfz.at[bi2], sem_s),
            pltpu.make_async_copy(fxn2, sfx.at[bj2], sem_s),
            pltpu.make_async_copy(fyn2, sfy.at[bj2], sem_s),
            pltpu.make_async_copy(fzn2, sfz.at[bj2], sem_s),
        ]

    def _pair(p, _):
        eb_a = ((2 * p) * NW + w) * CHUNK
        eb_b = ((2 * p + 1) * NW + w) * CHUNK
        ds_a = _phase(eb_a, bi, bj, fxp, fyp, fzp, fxn, fyn, fzn,
                      _prev_b_waits())
        ds_b = _phase(eb_b, bi2, bj2, fxp2, fyp2, fzp2, fxn2, fyn2, fzn2,
                      ds_a)
        return 0

    lax.fori_loop(0, n_pairs, _pair, 0)
    for d in _prev_b_waits():
        d.wait()

    plsc.subcore_barrier()
    for comp, sf in enumerate((sfx, sfy, sfz)):
        pltpu.sync_copy(sf.at[pl.ds(off, SSPAN)], zbuf)
        pltpu.sync_copy(zbuf, fpart.at[pl.ds((c * 3 + comp) * NPAD + off, SSPAN)])


_sc_call = functools.partial(
    pl.kernel,
    out_type=jax.ShapeDtypeStruct((6 * NPAD,), _f32),
    mesh=plsc.VectorSubcoreMesh(core_axis_name="c", subcore_axis_name="s"),
    scratch_types=[
        pltpu.VMEM((CHUNK,), _i32),
        pltpu.VMEM((CHUNK,), _i32),
        pltpu.VMEM((CHUNK,), _i32),
        pltpu.VMEM((CHUNK,), _i32),
        pltpu.VMEM((CHUNK,), _f32),
        pltpu.VMEM((CHUNK,), _f32),
        pltpu.VMEM((CHUNK,), _f32),
        pltpu.VMEM((CHUNK,), _f32),
        pltpu.VMEM((CHUNK,), _f32),
        pltpu.VMEM((CHUNK,), _f32),
        pltpu.VMEM((CHUNK,), _f32),
        pltpu.VMEM((CHUNK,), _f32),
        pltpu.VMEM((CHUNK,), _f32),
        pltpu.VMEM((CHUNK,), _f32),
        pltpu.VMEM((CHUNK,), _f32),
        pltpu.VMEM((CHUNK,), _f32),
        pltpu.VMEM((CHUNK,), _f32),
        pltpu.VMEM((CHUNK,), _f32),
        pltpu.VMEM((CHUNK,), _f32),
        pltpu.VMEM((CHUNK,), _f32),
        pltpu.VMEM((CHUNK,), _f32),
        pltpu.VMEM((CHUNK,), _f32),
        pltpu.VMEM((CHUNK,), _f32),
        pltpu.VMEM((CHUNK,), _f32),
        pltpu.VMEM((CHUNK,), _f32),
        pltpu.VMEM((CHUNK,), _f32),
        pltpu.VMEM((CHUNK,), _f32),
        pltpu.VMEM((CHUNK,), _f32),
        pltpu.VMEM((SSPAN,), _f32),
        pltpu.SemaphoreType.DMA,
        pltpu.SemaphoreType.DMA,
        pltpu.SemaphoreType.DMA,
        pltpu.VMEM_SHARED((NPAD,), _f32),
        pltpu.VMEM_SHARED((NPAD,), _f32),
        pltpu.VMEM_SHARED((NPAD,), _f32),
        pltpu.VMEM_SHARED((NPAD,), _f32),
        pltpu.VMEM_SHARED((NPAD,), _f32),
        pltpu.VMEM_SHARED((NPAD,), _f32),
    ],
)(_sc_body)


def _loss_body(fp_ref, ll_ref, bc_ref, out_ref):
    r = fp_ref[0] + fp_ref[1] + ll_ref[...]
    free = bc_ref[...] < 0.5
    free3 = jnp.broadcast_to(free, r.shape)
    masked = jnp.where(free3, r * r, jnp.zeros_like(r))
    nfree = jnp.sum(jnp.where(free, 1.0, 0.0).astype(_f32))
    out_ref[0, 0] = jnp.sum(masked) / (nfree * 3.0)


_loss_call = pl.pallas_call(
    _loss_body,
    out_shape=jax.ShapeDtypeStruct((1, 1), _f32),
    out_specs=pl.BlockSpec(memory_space=pltpu.SMEM),
)


def kernel(pred, connectivity, elem_directions, elem_lengths, prop_E, prop_A,
           line_load, bc_disp):
    padn = NPAD - N_NODES
    ux = jnp.pad(pred[:, 0], (0, padn))
    uy = jnp.pad(pred[:, 1], (0, padn))
    uz = jnp.pad(pred[:, 2], (0, padn))
    conn = connectivity.astype(_i32)
    ci = conn[:, 0]
    cj = conn[:, 1]
    dx = elem_directions[:, 0]
    dy = elem_directions[:, 1]
    dz = elem_directions[:, 2]
    fpart = _sc_call(ux, uy, uz, ci, cj, dx, dy, dz, prop_E, prop_A,
                     elem_lengths).reshape(2, 3, NPAD)
    llt = jnp.pad(line_load.T, ((0, 0), (0, padn)))
    bcp = jnp.pad(bc_disp[:, 0], (0, padn), constant_values=1.0).reshape(1, NPAD)
    loss2d = _loss_call(fpart, llt, bcp)
    return loss2d[0, 0]
